# baseline (speedup 1.0000x reference)
import jax
import jax.numpy as jnp
from jax.experimental import pallas as pl
from jax.experimental.pallas import tpu as pltpu

N = 100000
NFEAT = 512
NHID = 1024
NCLASS = 256
BM = 5000
SUB = 10
TM = BM // SUB


def _mlp_kernel(x_ref, w1_ref, b1_ref, w2_ref, b2_ref, o_ref):
    for t in range(SUB):
        rows = pl.ds(t * TM, TM)
        xb = x_ref[rows, :]
        h = jax.lax.dot_general(
            xb, w1_ref[:],
            dimension_numbers=(((1,), (0,)), ((), ())),
            preferred_element_type=jnp.float32,
            precision=jax.lax.Precision.DEFAULT,
        )
        h = jnp.maximum(h + b1_ref[:], 0.0)
        out = jax.lax.dot_general(
            h, w2_ref[:],
            dimension_numbers=(((1,), (0,)), ((), ())),
            preferred_element_type=jnp.float32,
            precision=jax.lax.Precision.DEFAULT,
        )
        out = out + b2_ref[:]
        m = jnp.max(out, axis=1, keepdims=True)
        s = out - m
        lse = jnp.log(jnp.sum(jnp.exp(s), axis=1, keepdims=True))
        o_ref[rows, :] = s - lse


def kernel(x, W1, b1, W2, b2):
    w1 = W1.astype(jnp.bfloat16)
    w2 = W2.astype(jnp.bfloat16)
    b1r = b1.reshape(1, NHID)
    b2r = b2.reshape(1, NCLASS)
    return pl.pallas_call(
        _mlp_kernel,
        grid=(N // BM,),
        in_specs=[
            pl.BlockSpec((BM, NFEAT), lambda i: (i, 0)),
            pl.BlockSpec((NFEAT, NHID), lambda i: (0, 0)),
            pl.BlockSpec((1, NHID), lambda i: (0, 0)),
            pl.BlockSpec((NHID, NCLASS), lambda i: (0, 0)),
            pl.BlockSpec((1, NCLASS), lambda i: (0, 0)),
        ],
        out_specs=pl.BlockSpec((BM, NCLASS), lambda i: (i, 0)),
        out_shape=jax.ShapeDtypeStruct((N, NCLASS), jnp.float32),
        compiler_params=pltpu.CompilerParams(
            dimension_semantics=("arbitrary",),
        ),
    )(x, w1, b1r, w2, b2r)


# speedup vs baseline: 1.0077x; 1.0077x over previous
import jax
import jax.numpy as jnp
from jax.experimental import pallas as pl
from jax.experimental.pallas import tpu as pltpu

N = 100000
NFEAT = 512
NHID = 1024
NCLASS = 256
BM = 5000
SUB = 5
TM = BM // SUB


def _mlp_kernel(x_ref, w1_ref, b1_ref, w2_ref, b2_ref, o_ref):
    for t in range(SUB):
        rows = pl.ds(t * TM, TM)
        xb = x_ref[rows, :]
        h = jax.lax.dot_general(
            xb, w1_ref[:],
            dimension_numbers=(((1,), (0,)), ((), ())),
            preferred_element_type=jnp.float32,
            precision=jax.lax.Precision.DEFAULT,
        )
        h = jnp.maximum(h + b1_ref[:], 0.0)
        out = jax.lax.dot_general(
            h, w2_ref[:],
            dimension_numbers=(((1,), (0,)), ((), ())),
            preferred_element_type=jnp.float32,
            precision=jax.lax.Precision.DEFAULT,
        )
        out = out + b2_ref[:]
        m = jnp.max(out, axis=1, keepdims=True)
        s = out - m
        lse = jnp.log(jnp.sum(jnp.exp(s), axis=1, keepdims=True))
        o_ref[rows, :] = s - lse


def kernel(x, W1, b1, W2, b2):
    w1 = W1.astype(jnp.bfloat16)
    w2 = W2.astype(jnp.bfloat16)
    b1r = b1.reshape(1, NHID)
    b2r = b2.reshape(1, NCLASS)
    return pl.pallas_call(
        _mlp_kernel,
        grid=(N // BM,),
        in_specs=[
            pl.BlockSpec((BM, NFEAT), lambda i: (i, 0)),
            pl.BlockSpec((NFEAT, NHID), lambda i: (0, 0)),
            pl.BlockSpec((1, NHID), lambda i: (0, 0)),
            pl.BlockSpec((NHID, NCLASS), lambda i: (0, 0)),
            pl.BlockSpec((1, NCLASS), lambda i: (0, 0)),
        ],
        out_specs=pl.BlockSpec((BM, NCLASS), lambda i: (i, 0)),
        out_shape=jax.ShapeDtypeStruct((N, NCLASS), jnp.float32),
        compiler_params=pltpu.CompilerParams(
            dimension_semantics=("arbitrary",),
        ),
    )(x, w1, b1r, w2, b2r)
